# baseline (speedup 1.0000x reference)
import functools

import jax
import jax.numpy as jnp
from jax import lax
from jax.experimental import pallas as pl
from jax.experimental.pallas import tpu as pltpu
from jax.experimental.pallas import tpu_sc as plsc


@functools.lru_cache(maxsize=None)
def _build_gather(batch: int, seq: int, vocab: int, dim: int):
    info = plsc.get_sparse_core_info()
    nc, ns = info.num_cores, info.num_subcores
    nw = nc * ns
    nbuf = 3
    per_w = batch // nw
    sm = seq - seq % 8
    tr = seq - sm
    tw = per_w * tr
    assert batch % nw == 0 and sm > 0 and tw % 8 == 0

    mesh = plsc.VectorSubcoreMesh(core_axis_name="c", subcore_axis_name="s")

    @functools.partial(
        pl.kernel,
        mesh=mesh,
        out_type=(
            jax.ShapeDtypeStruct((batch, seq, dim), jnp.float32),
            jax.ShapeDtypeStruct((batch * tr, dim), jnp.float32),
        ),
        scratch_types=[
            pltpu.VMEM((per_w, sm), jnp.int32),
            pltpu.VMEM((1, tw), jnp.int32),
            pltpu.VMEM((tw, dim), jnp.float32),
        ]
        + [pltpu.VMEM((sm, dim), jnp.float32)] * nbuf
        + [pltpu.SemaphoreType.DMA] * (2 * nbuf + 2),
    )
    def gather_kernel(idx_hbm, tidx_hbm, table_hbm, out_hbm, tail_hbm,
                      idx_v, tidx_v, tbuf, *bufs_and_sems):
        rows = bufs_and_sems[:nbuf]
        gsems = bufs_and_sems[nbuf:2 * nbuf]
        osems = bufs_and_sems[2 * nbuf:3 * nbuf]
        tsem_g, tsem_o = bufs_and_sems[3 * nbuf:]
        wid = lax.axis_index("s") * nc + lax.axis_index("c")
        ebase = wid * per_w
        pltpu.sync_copy(idx_hbm.at[wid], idx_v)
        pltpu.sync_copy(tidx_hbm.at[wid], tidx_v)
        tail_g = pltpu.async_copy(table_hbm.at[tidx_v.at[0]], tbuf, tsem_g)
        gathers = {}
        outs = {}
        for j in range(nbuf - 1):
            gathers[j] = pltpu.async_copy(
                table_hbm.at[idx_v.at[j]], rows[j], gsems[j])
        for j in range(per_w):
            b = j % nbuf
            jn = j + nbuf - 1
            if jn < per_w:
                bn = jn % nbuf
                if jn - nbuf >= 0:
                    outs[jn - nbuf].wait()
                gathers[jn] = pltpu.async_copy(
                    table_hbm.at[idx_v.at[jn]], rows[bn], gsems[bn])
            gathers[j].wait()
            outs[j] = pltpu.async_copy(
                rows[b], out_hbm.at[ebase + j, pl.ds(0, sm)], osems[b])
            if j == per_w // 2:
                tail_g.wait()
                tail_o = pltpu.async_copy(
                    tbuf, tail_hbm.at[pl.ds(wid * tw, tw)], tsem_o)
        for j in range(per_w - nbuf, per_w):
            outs[j].wait()
        tail_o.wait()

    return gather_kernel


def kernel(indices, table):
    batch, seq = indices.shape
    vocab, dim = table.shape
    gather = _build_gather(batch, seq, vocab, dim)
    info = plsc.get_sparse_core_info()
    nw = info.num_cores * info.num_subcores
    per_w = batch // nw
    sm = seq - seq % 8
    tr = seq - sm
    idx3 = indices.reshape(nw, per_w, seq).astype(jnp.int32)
    idx_main = idx3[:, :, :sm]
    idx_tail = idx3[:, :, sm:].reshape(nw, 1, per_w * tr)
    out, tail = gather(idx_main, idx_tail, table)
    zero = lax.optimization_barrier(jnp.float32(0.0))
    tail3 = tail.reshape(batch, tr, dim) + zero
    return lax.dynamic_update_slice(out, tail3, (0, sm, 0))
